# baseline (speedup 1.0000x reference)
import functools

import jax
import jax.numpy as jnp
from jax.experimental import pallas as pl
from jax.experimental.pallas import tpu as pltpu

_ANCHORS_W = (10.0, 23.0, 37.0, 81.0, 135.0, 344.0)
_ANCHORS_H = (14.0, 27.0, 58.0, 82.0, 169.0, 319.0)
_NO_OBJECT_COEFF = 0.5
_COORD_COEFF = 5.0
_SMALL_OFFSET = 507.0
_MB = 4


def _softplus(x):
    return jnp.maximum(x, 0.0) + jnp.log1p(jnp.exp(-jnp.abs(x)))


def _bce(x, z):
    return jnp.maximum(x, 0.0) - x * z + jnp.log1p(jnp.exp(-jnp.abs(x)))


def _one_batch(nt, tgt, blk, wa, ha, N, T, A):
    txc = tgt[:, 0:1]
    tyc = tgt[:, 1:2]
    twc = tgt[:, 2:3]
    thc = tgt[:, 3:4]

    sub_t = jax.lax.broadcasted_iota(jnp.int32, (T, 1), 0)
    validb = sub_t < jnp.minimum(nt, T)

    inter_a = jnp.minimum(wa, twc) * jnp.minimum(ha, thc)
    iou_at = inter_a / (wa * ha + twc * thc - inter_a + 1e-09)
    m = jnp.max(iou_at, axis=1, keepdims=True)
    lane6 = jax.lax.broadcasted_iota(jnp.int32, (T, 6), 1)
    aidx = jnp.min(jnp.where(iou_at == m, lane6, 6), axis=1, keepdims=True)

    small = aidx < 3
    rstride = jnp.where(small, 1.0 / 16.0, 1.0 / 32.0)
    grid = jnp.where(small, 26.0, 13.0)
    xs = txc * rstride
    ys = tyc * rstride
    cx = jnp.floor(xs)
    cy = jnp.floor(ys)
    fx = jnp.clip(xs - cx, 1e-09, 1.0 - 1e-09)
    fy = jnp.clip(ys - cy, 1e-09, 1.0 - 1e-09)
    onehot_a = (lane6 == aidx).astype(jnp.float32)
    chosen_w = jnp.sum(onehot_a * wa, axis=1, keepdims=True)
    chosen_h = jnp.sum(onehot_a * ha, axis=1, keepdims=True)
    packed = jnp.concatenate(
        [fx, 1.0 - fx, fy, 1.0 - fy, twc / chosen_w, thc / chosen_h,
         jnp.ones_like(fx), jnp.ones_like(fx)], axis=1)
    lg = jnp.log(packed)
    tx = lg[:, 0:1] - lg[:, 1:2]
    ty = lg[:, 2:3] - lg[:, 3:4]
    tw = lg[:, 4:5]
    th = lg[:, 5:6]
    amod = (aidx - jnp.where(small, 0, 3)).astype(jnp.float32)
    lsm = small.astype(jnp.float32)
    obj_index = (lsm * _SMALL_OFFSET + grid * grid * amod
                 + grid * cy + cx)

    blkT = jnp.swapaxes(blk[:, 0:8], 0, 1)
    pxr = blkT[0:1, :]
    pyr = blkT[1:2, :]
    pwr = blkT[2:3, :]
    phr = blkT[3:4, :]
    pconf = blkT[4:5, :]

    px1 = pxr - pwr * 0.5
    py1 = pyr - phr * 0.5
    px2 = px1 + pwr
    py2 = py1 + phr
    aPe = pwr * phr + 1e-09
    tx1 = txc - twc * 0.5
    ty1 = tyc - thc * 0.5
    tx2 = tx1 + twc
    ty2 = ty1 + thc
    aT = twc * thc

    wI = jnp.maximum(jnp.minimum(px2, tx2) - jnp.maximum(px1, tx1), 0.0)
    hI = jnp.maximum(jnp.minimum(py2, ty2) - jnp.maximum(py1, ty1), 0.0)
    inter = wI * hI
    S = aPe + aT
    ge = jnp.logical_and(3.0 * inter >= S, S > inter)

    lane_n = jax.lax.broadcasted_iota(jnp.int32, (T, N), 1)
    eq = lane_n == obj_index.astype(jnp.int32)
    ohTN = jnp.where(jnp.logical_and(eq, validb), 1.0, 0.0)
    covf = jnp.maximum(jnp.where(jnp.logical_and(ge, validb), 1.0, 0.0), ohTN)
    covered = jnp.max(covf, axis=0, keepdims=True)
    noobj = jnp.sum(jnp.where(covered == 0.0, _softplus(pconf), 0.0))

    pobj = jax.lax.dot_general(ohTN, blk, (((1,), (0,)), ((), ())),
                               preferred_element_type=jnp.float32)

    lane_a = jax.lax.broadcasted_iota(jnp.int32, (T, A), 1)
    tgt_full = jnp.where(lane_a == 0, tx,
               jnp.where(lane_a == 1, ty,
               jnp.where(lane_a == 2, tw,
               jnp.where(lane_a == 3, th, tgt))))

    diff = pobj - tgt_full
    coord = jnp.sum(jnp.where(jnp.logical_and(lane_a < 4, validb), diff * diff, 0.0))
    E = _bce(pobj, tgt_full)
    objl = jnp.sum(jnp.where(jnp.logical_and(lane_a == 4, validb), E, 0.0))
    clsl = jnp.sum(jnp.where(jnp.logical_and(lane_a >= 5, validb), E, 0.0))
    return coord, objl, noobj, clsl


def _group_kernel(nt_ref, pred_ref, tgt_ref, anch_ref, part_ref, *, N, T, A):
    g = pl.program_id(0)
    wa = anch_ref[0:1, :]
    ha = anch_ref[1:2, :]
    acc = [0.0, 0.0, 0.0, 0.0]
    for i in range(_MB):
        nt = nt_ref[g * _MB + i]
        res = _one_batch(nt, tgt_ref[i], pred_ref[i], wa, ha, N, T, A)
        for k in range(4):
            acc[k] = acc[k] + res[k]
    for k in range(4):
        part_ref[0, 0, k] = acc[k]


def _reduce_kernel(part_ref, total_ref, coord_ref, obj_ref, noobj_ref, class_ref):
    p = part_ref[:, 0, :]
    s = jnp.sum(p, axis=0, keepdims=True)
    coord = s[0, 0]
    objl = s[0, 1]
    noobj = s[0, 2]
    clsl = s[0, 3]
    coord_ref[0, 0] = coord
    obj_ref[0, 0] = objl
    noobj_ref[0, 0] = noobj
    class_ref[0, 0] = clsl
    total_ref[0, 0] = (clsl + objl + _COORD_COEFF * coord
                       + _NO_OBJECT_COEFF * noobj)


def kernel(predictions, targets, num_targets):
    B, N, A = predictions.shape
    T = targets.shape[1]
    G = B // _MB
    anchors = jnp.asarray([_ANCHORS_W, _ANCHORS_H], dtype=jnp.float32)

    smem_spec = pl.BlockSpec(memory_space=pltpu.SMEM)
    partials = pl.pallas_call(
        functools.partial(_group_kernel, N=N, T=T, A=A),
        grid=(G,),
        in_specs=[
            smem_spec,
            pl.BlockSpec((_MB, N, A), lambda g: (g, 0, 0)),
            pl.BlockSpec((_MB, T, A), lambda g: (g, 0, 0)),
            pl.BlockSpec((2, 6), lambda g: (0, 0)),
        ],
        out_specs=pl.BlockSpec((1, 1, 4), lambda g: (g, 0, 0), memory_space=pltpu.SMEM),
        out_shape=jax.ShapeDtypeStruct((G, 1, 4), jnp.float32),
        compiler_params=pltpu.CompilerParams(
            dimension_semantics=("parallel",)),
    )(num_targets, predictions, targets, anchors)

    outs = pl.pallas_call(
        _reduce_kernel,
        out_specs=[smem_spec] * 5,
        out_shape=[jax.ShapeDtypeStruct((1, 1), jnp.float32) for _ in range(5)],
    )(partials)
    total, coord, obj, noobj, cls = [o[0, 0] for o in outs]
    return (total, coord, obj, noobj, cls)


# speedup vs baseline: 1.0324x; 1.0324x over previous
import functools

import jax
import jax.numpy as jnp
from jax.experimental import pallas as pl
from jax.experimental.pallas import tpu as pltpu

_ANCHORS_W = (10.0, 23.0, 37.0, 81.0, 135.0, 344.0)
_ANCHORS_H = (14.0, 27.0, 58.0, 82.0, 169.0, 319.0)
_NO_OBJECT_COEFF = 0.5
_COORD_COEFF = 5.0
_SMALL_OFFSET = 507.0


def _softplus(x):
    return jnp.maximum(x, 0.0) + jnp.log1p(jnp.exp(-jnp.abs(x)))


def _bce(x, z):
    return jnp.maximum(x, 0.0) - x * z + jnp.log1p(jnp.exp(-jnp.abs(x)))


def _one_batch(nt, tgt, blk, wa, ha, N, T, A):
    txc = tgt[:, 0:1]
    tyc = tgt[:, 1:2]
    twc = tgt[:, 2:3]
    thc = tgt[:, 3:4]

    sub_t = jax.lax.broadcasted_iota(jnp.int32, (T, 1), 0)
    validb = sub_t < jnp.minimum(nt, T)

    inter_a = jnp.minimum(wa, twc) * jnp.minimum(ha, thc)
    iou_at = inter_a / (wa * ha + twc * thc - inter_a + 1e-09)
    m = jnp.max(iou_at, axis=1, keepdims=True)
    lane6 = jax.lax.broadcasted_iota(jnp.int32, (T, 6), 1)
    aidx = jnp.min(jnp.where(iou_at == m, lane6, 6), axis=1, keepdims=True)

    small = aidx < 3
    rstride = jnp.where(small, 1.0 / 16.0, 1.0 / 32.0)
    grid = jnp.where(small, 26.0, 13.0)
    xs = txc * rstride
    ys = tyc * rstride
    cx = jnp.floor(xs)
    cy = jnp.floor(ys)
    fx = jnp.clip(xs - cx, 1e-09, 1.0 - 1e-09)
    fy = jnp.clip(ys - cy, 1e-09, 1.0 - 1e-09)
    onehot_a = (lane6 == aidx).astype(jnp.float32)
    chosen_w = jnp.sum(onehot_a * wa, axis=1, keepdims=True)
    chosen_h = jnp.sum(onehot_a * ha, axis=1, keepdims=True)
    packed = jnp.concatenate(
        [fx, 1.0 - fx, fy, 1.0 - fy, twc / chosen_w, thc / chosen_h,
         jnp.ones_like(fx), jnp.ones_like(fx)], axis=1)
    lg = jnp.log(packed)
    tx = lg[:, 0:1] - lg[:, 1:2]
    ty = lg[:, 2:3] - lg[:, 3:4]
    tw = lg[:, 4:5]
    th = lg[:, 5:6]
    amod = (aidx - jnp.where(small, 0, 3)).astype(jnp.float32)
    lsm = small.astype(jnp.float32)
    obj_index = (lsm * _SMALL_OFFSET + grid * grid * amod
                 + grid * cy + cx)

    blkT = jnp.swapaxes(blk[:, 0:8], 0, 1)
    pxr = blkT[0:1, :]
    pyr = blkT[1:2, :]
    pwr = blkT[2:3, :]
    phr = blkT[3:4, :]
    pconf = blkT[4:5, :]

    px1 = pxr - pwr * 0.5
    py1 = pyr - phr * 0.5
    px2 = px1 + pwr
    py2 = py1 + phr
    aPe = pwr * phr + 1e-09
    tx1 = txc - twc * 0.5
    ty1 = tyc - thc * 0.5
    tx2 = tx1 + twc
    ty2 = ty1 + thc
    aT = twc * thc

    wI = jnp.maximum(jnp.minimum(px2, tx2) - jnp.maximum(px1, tx1), 0.0)
    hI = jnp.maximum(jnp.minimum(py2, ty2) - jnp.maximum(py1, ty1), 0.0)
    inter = wI * hI
    S = aPe + aT
    ge = jnp.logical_and(3.0 * inter >= S, S > inter)

    lane_n = jax.lax.broadcasted_iota(jnp.int32, (T, N), 1)
    eq = lane_n == obj_index.astype(jnp.int32)
    ohTN = jnp.where(jnp.logical_and(eq, validb), 1.0, 0.0)
    covf = jnp.maximum(jnp.where(jnp.logical_and(ge, validb), 1.0, 0.0), ohTN)
    covered = jnp.max(covf, axis=0, keepdims=True)
    noobj = jnp.sum(jnp.where(covered == 0.0, _softplus(pconf), 0.0))

    pobj = jax.lax.dot_general(ohTN, blk, (((1,), (0,)), ((), ())),
                               preferred_element_type=jnp.float32)

    lane_a = jax.lax.broadcasted_iota(jnp.int32, (T, A), 1)
    tgt_full = jnp.where(lane_a == 0, tx,
               jnp.where(lane_a == 1, ty,
               jnp.where(lane_a == 2, tw,
               jnp.where(lane_a == 3, th, tgt))))

    diff = pobj - tgt_full
    coord = jnp.sum(jnp.where(jnp.logical_and(lane_a < 4, validb), diff * diff, 0.0))
    E = _bce(pobj, tgt_full)
    objl = jnp.sum(jnp.where(jnp.logical_and(lane_a == 4, validb), E, 0.0))
    clsl = jnp.sum(jnp.where(jnp.logical_and(lane_a >= 5, validb), E, 0.0))
    return coord, objl, noobj, clsl


def _loss_kernel(nt_ref, pred_ref, tgt_ref, anch_ref,
                 total_ref, coord_ref, obj_ref, noobj_ref, class_ref,
                 *, B, N, T, A):
    b = pl.program_id(0)

    @pl.when(b == 0)
    def _init():
        total_ref[0, 0] = 0.0
        coord_ref[0, 0] = 0.0
        obj_ref[0, 0] = 0.0
        noobj_ref[0, 0] = 0.0
        class_ref[0, 0] = 0.0

    wa = anch_ref[0:1, :]
    ha = anch_ref[1:2, :]
    coord, objl, noobj, clsl = _one_batch(
        nt_ref[b], tgt_ref[0], pred_ref[0], wa, ha, N, T, A)
    coord_ref[0, 0] += coord
    obj_ref[0, 0] += objl
    noobj_ref[0, 0] += noobj
    class_ref[0, 0] += clsl

    @pl.when(b == B - 1)
    def _fin():
        total_ref[0, 0] = (class_ref[0, 0] + obj_ref[0, 0]
                           + _COORD_COEFF * coord_ref[0, 0]
                           + _NO_OBJECT_COEFF * noobj_ref[0, 0])


def kernel(predictions, targets, num_targets):
    B, N, A = predictions.shape
    T = targets.shape[1]
    anchors = jnp.asarray([_ANCHORS_W, _ANCHORS_H], dtype=jnp.float32)

    smem_spec = pl.BlockSpec(memory_space=pltpu.SMEM)
    outs = pl.pallas_call(
        functools.partial(_loss_kernel, B=B, N=N, T=T, A=A),
        grid=(B,),
        in_specs=[
            smem_spec,
            pl.BlockSpec((1, N, A), lambda b: (b, 0, 0)),
            pl.BlockSpec((1, T, A), lambda b: (b, 0, 0)),
            pl.BlockSpec((2, 6), lambda b: (0, 0)),
        ],
        out_specs=[smem_spec] * 5,
        out_shape=[jax.ShapeDtypeStruct((1, 1), jnp.float32) for _ in range(5)],
    )(num_targets, predictions, targets, anchors)
    total, coord, obj, noobj, cls = [o[0, 0] for o in outs]
    return (total, coord, obj, noobj, cls)
